# baseline (speedup 1.0000x reference)
import functools
import jax
import jax.numpy as jnp
from jax import lax
from jax.experimental import pallas as pl
from jax.experimental.pallas import tpu as pltpu
from jax.experimental.pallas import tpu_sc as plsc

_L = 16
_G = 16
_CW = 1024


def _sc_scan_kernel(n_rows, n_cols, rows_per_worker, x_hbm, o_hbm,
                    in_ref, out_ref, fin_ref, in_sems, out_sems):
    wid = lax.axis_index("c") * 16 + lax.axis_index("s")
    n_passes = rows_per_worker // _G
    n_chunks = n_cols // _CW
    n_vecs = _CW // _L
    n_total = n_passes * n_chunks
    iota = lax.iota(jnp.int32, _L)
    lane15 = jnp.full((_L,), _L - 1, jnp.int32)
    zidx = jnp.zeros((_L,), jnp.int32)
    row_base = wid * rows_per_worker

    def in_copy(k, buf):
        r0 = row_base + (k // n_chunks) * _G
        c0 = (k % n_chunks) * _CW
        return pltpu.make_async_copy(
            x_hbm.at[pl.ds(r0, _G), pl.ds(c0, _CW)],
            in_ref.at[buf], in_sems.at[buf])

    def out_copy(k, buf):
        r0 = row_base + (k // n_chunks) * _G
        c0 = (k % n_chunks) * _CW
        return pltpu.make_async_copy(
            out_ref.at[buf],
            o_hbm.at[pl.ds(r0, _G), pl.ds(c0, _CW)], out_sems.at[buf])

    in_copy(0, 0).start()

    def step(k, carries):
        buf = lax.rem(k, 2)
        first = lax.rem(k, n_chunks) == 0
        carries = tuple(
            jnp.where(first, jnp.zeros((_L,), jnp.float32), c)
            for c in carries)

        in_copy(k, buf).wait()

        @pl.when(k + 1 < n_total)
        def _():
            in_copy(k + 1, 1 - buf).start()

        @pl.when(k >= 1)
        def _():
            out_copy(k - 1, 1 - buf).wait()

        ib = in_ref.at[buf]
        ob = out_ref.at[buf]

        @plsc.parallel_loop(0, n_vecs, carry=carries)
        def vec_loop(i, carries):
            new = []
            for g in range(_G):
                v = ib[g, pl.ds(i * _L, _L)]
                s = plsc.cumsum(v)
                ob[g, pl.ds(i * _L, _L)] = s - v + carries[g]
                new.append(carries[g] + jnp.take(s, lane15))
            return tuple(new)

        carries = vec_loop
        out_copy(k, buf).start()

        @pl.when(lax.rem(k, n_chunks) == n_chunks - 1)
        def _():
            for g in range(_G):
                plsc.store_scatter(
                    fin_ref, [zidx + g, zidx], carries[g], mask=iota == 0)
            r0 = row_base + (k // n_chunks) * _G
            fin_c0 = r0 * 0 + n_cols
            pltpu.sync_copy(
                fin_ref, o_hbm.at[pl.ds(r0, _G), pl.ds(fin_c0, 128)])

        return carries

    zero = jnp.zeros((_L,), jnp.float32)
    lax.fori_loop(0, n_total, step, (zero,) * _G)
    out_copy(n_total - 1, lax.rem(n_total - 1, 2)).wait()


def _exclusive_scan_sc(x):
    n_rows, n_cols = x.shape
    n_workers = 32
    rows_per_worker = n_rows // n_workers
    mesh = plsc.VectorSubcoreMesh(core_axis_name="c", subcore_axis_name="s")
    kern = pl.kernel(
        functools.partial(_sc_scan_kernel, n_rows, n_cols, rows_per_worker),
        out_type=jax.ShapeDtypeStruct((n_rows - 1, n_cols + 1), x.dtype),
        mesh=mesh,
        compiler_params=pltpu.CompilerParams(needs_layout_passes=False),
        scratch_types=[
            pltpu.VMEM((2, _G, _CW), jnp.float32),
            pltpu.VMEM((2, _G, _CW), jnp.float32),
            pltpu.VMEM((_G, 128), jnp.float32),
            pltpu.SemaphoreType.DMA((2,)),
            pltpu.SemaphoreType.DMA((2,)),
        ],
    )
    return kern(x)


def kernel(x):
    return _exclusive_scan_sc(x)


# speedup vs baseline: 1.0185x; 1.0185x over previous
import functools
import jax
import jax.numpy as jnp
from jax import lax
from jax.experimental import pallas as pl
from jax.experimental.pallas import tpu as pltpu
from jax.experimental.pallas import tpu_sc as plsc

_L = 16
_G = 8
_CW = 2048


def _sc_scan_kernel(n_rows, n_cols, rows_per_worker, x_hbm, o_hbm,
                    in_ref, out_ref, fin_ref, in_sems, out_sems):
    wid = lax.axis_index("c") * 16 + lax.axis_index("s")
    n_passes = rows_per_worker // _G
    n_chunks = n_cols // _CW
    n_vecs = _CW // _L
    n_total = n_passes * n_chunks
    iota = lax.iota(jnp.int32, _L)
    lane15 = jnp.full((_L,), _L - 1, jnp.int32)
    zidx = jnp.zeros((_L,), jnp.int32)
    row_base = wid * rows_per_worker

    def in_copy(k, buf):
        r0 = row_base + (k // n_chunks) * _G
        c0 = (k % n_chunks) * _CW
        return pltpu.make_async_copy(
            x_hbm.at[pl.ds(r0, _G), pl.ds(c0, _CW)],
            in_ref.at[buf], in_sems.at[buf])

    def out_copy(k, buf):
        r0 = row_base + (k // n_chunks) * _G
        c0 = (k % n_chunks) * _CW
        return pltpu.make_async_copy(
            out_ref.at[buf],
            o_hbm.at[pl.ds(r0, _G), pl.ds(c0, _CW)], out_sems.at[buf])

    in_copy(0, 0).start()

    def step(k, carries):
        buf = lax.rem(k, 2)
        first = lax.rem(k, n_chunks) == 0
        carries = tuple(
            jnp.where(first, jnp.zeros((_L,), jnp.float32), c)
            for c in carries)

        in_copy(k, buf).wait()

        @pl.when(k + 1 < n_total)
        def _():
            in_copy(k + 1, 1 - buf).start()

        @pl.when(k >= 1)
        def _():
            out_copy(k - 1, 1 - buf).wait()

        ib = in_ref.at[buf]
        ob = out_ref.at[buf]

        @plsc.parallel_loop(0, n_vecs, carry=carries)
        def vec_loop(i, carries):
            new = []
            for g in range(_G):
                v = ib[g, pl.ds(i * _L, _L)]
                s = plsc.cumsum(v)
                ob[g, pl.ds(i * _L, _L)] = s - v + carries[g]
                new.append(carries[g] + jnp.take(s, lane15))
            return tuple(new)

        carries = vec_loop
        out_copy(k, buf).start()

        @pl.when(lax.rem(k, n_chunks) == n_chunks - 1)
        def _():
            for g in range(_G):
                plsc.store_scatter(
                    fin_ref, [zidx + g, zidx], carries[g], mask=iota == 0)
            r0 = row_base + (k // n_chunks) * _G
            fin_c0 = r0 * 0 + n_cols
            pltpu.sync_copy(
                fin_ref, o_hbm.at[pl.ds(r0, _G), pl.ds(fin_c0, 128)])

        return carries

    zero = jnp.zeros((_L,), jnp.float32)
    lax.fori_loop(0, n_total, step, (zero,) * _G)
    out_copy(n_total - 1, lax.rem(n_total - 1, 2)).wait()


def _exclusive_scan_sc(x):
    n_rows, n_cols = x.shape
    n_workers = 32
    rows_per_worker = n_rows // n_workers
    mesh = plsc.VectorSubcoreMesh(core_axis_name="c", subcore_axis_name="s")
    kern = pl.kernel(
        functools.partial(_sc_scan_kernel, n_rows, n_cols, rows_per_worker),
        out_type=jax.ShapeDtypeStruct((n_rows - 1, n_cols + 1), x.dtype),
        mesh=mesh,
        compiler_params=pltpu.CompilerParams(needs_layout_passes=False),
        scratch_types=[
            pltpu.VMEM((2, _G, _CW), jnp.float32),
            pltpu.VMEM((2, _G, _CW), jnp.float32),
            pltpu.VMEM((_G, 128), jnp.float32),
            pltpu.SemaphoreType.DMA((2,)),
            pltpu.SemaphoreType.DMA((2,)),
        ],
    )
    return kern(x)


def kernel(x):
    return _exclusive_scan_sc(x)


# speedup vs baseline: 1.1701x; 1.1488x over previous
import functools
import jax
import jax.numpy as jnp
from jax import lax
from jax.experimental import pallas as pl
from jax.experimental.pallas import tpu as pltpu
from jax.experimental.pallas import tpu_sc as plsc

_L = 16
_G = 8
_CW = 4096
_OW = 2048


def _sc_scan_kernel(n_rows, n_cols, rows_per_worker, x_hbm, o_hbm,
                    in_ref, out_ref, fin_ref, in_sems, out_sems):
    wid = lax.axis_index("c") * 16 + lax.axis_index("s")
    n_passes = rows_per_worker // _G
    n_chunks = n_cols // _CW
    n_halves = _CW // _OW
    n_vecs = _OW // _L
    n_total = n_passes * n_chunks
    iota = lax.iota(jnp.int32, _L)
    lane15 = jnp.full((_L,), _L - 1, jnp.int32)
    zidx = jnp.zeros((_L,), jnp.int32)
    row_base = wid * rows_per_worker

    def in_copy(k, buf):
        r0 = row_base + (k // n_chunks) * _G
        c0 = (k % n_chunks) * _CW
        return pltpu.make_async_copy(
            x_hbm.at[pl.ds(r0, _G), pl.ds(c0, _CW)],
            in_ref.at[buf], in_sems.at[buf])

    def out_copy(k, h):
        r0 = row_base + (k // n_chunks) * _G
        c0 = (k % n_chunks) * _CW + h * _OW
        return pltpu.make_async_copy(
            out_ref.at[h],
            o_hbm.at[pl.ds(r0, _G), pl.ds(c0, _OW)], out_sems.at[h])

    in_copy(0, 0).start()

    def pass_body(p, _):
      def step(c, carries):
        k = p * n_chunks + c
        buf = lax.rem(k, 2)

        in_copy(k, buf).wait()

        @pl.when(k + 1 < n_total)
        def _():
            in_copy(k + 1, 1 - buf).start()

        ib = in_ref.at[buf]

        for h in range(n_halves):
            @pl.when(k >= 1)
            def _():
                out_copy(k - 1, h).wait()

            ob = out_ref.at[h]

            @plsc.parallel_loop(0, n_vecs, carry=carries)
            def vec_loop(i, carries):
                new = []
                for g in range(_G):
                    v = ib[g, pl.ds(h * _OW + i * _L, _L)]
                    s = plsc.cumsum(v)
                    ob[g, pl.ds(i * _L, _L)] = s - v + carries[g]
                    new.append(carries[g] + jnp.take(s, lane15))
                return tuple(new)

            carries = vec_loop
            out_copy(k, h).start()

        return carries

      zero = jnp.zeros((_L,), jnp.float32)
      carries = lax.fori_loop(0, n_chunks, step, (zero,) * _G)

      for g in range(_G):
          plsc.store_scatter(
              fin_ref, [zidx + g, zidx], carries[g], mask=iota == 0)
      r0 = row_base + p * _G
      fin_c0 = r0 * 0 + n_cols
      pltpu.sync_copy(
          fin_ref, o_hbm.at[pl.ds(r0, _G), pl.ds(fin_c0, 128)])
      return 0

    lax.fori_loop(0, n_passes, pass_body, 0)
    for h in range(n_halves):
        out_copy(n_total - 1, h).wait()


def _exclusive_scan_sc(x):
    n_rows, n_cols = x.shape
    n_workers = 32
    rows_per_worker = n_rows // n_workers
    mesh = plsc.VectorSubcoreMesh(core_axis_name="c", subcore_axis_name="s")
    kern = pl.kernel(
        functools.partial(_sc_scan_kernel, n_rows, n_cols, rows_per_worker),
        out_type=jax.ShapeDtypeStruct((n_rows - 1, n_cols + 1), x.dtype),
        mesh=mesh,
        compiler_params=pltpu.CompilerParams(needs_layout_passes=False),
        scratch_types=[
            pltpu.VMEM((2, _G, _CW), jnp.float32),
            pltpu.VMEM((2, _G, _OW), jnp.float32),
            pltpu.VMEM((_G, 128), jnp.float32),
            pltpu.SemaphoreType.DMA((2,)),
            pltpu.SemaphoreType.DMA((2,)),
        ],
    )
    return kern(x)


def kernel(x):
    return _exclusive_scan_sc(x)
